# baseline (speedup 1.0000x reference)
import functools
import math

import jax
import jax.numpy as jnp
from jax import lax
from jax.experimental import pallas as pl
from jax.experimental.pallas import tpu as pltpu
from jax.experimental.pallas import tpu_sc as plsc

DIM = 64
HID = 128
G = 128
BN = 4096


def _xpose_body(t_ref, o_ref):
    x = t_ref[...]
    y = jnp.transpose(x, (1, 0))
    o_ref[...] = jnp.concatenate([y[:BN // 2], y[BN // 2:]], axis=1)


def _transpose_pack(tabT):
    d, n = tabT.shape
    nb = (n + BN - 1) // BN
    return pl.pallas_call(
        _xpose_body,
        grid=(nb,),
        in_specs=[pl.BlockSpec((64, BN), lambda i: (0, i))],
        out_specs=pl.BlockSpec((BN // 2, 2 * DIM), lambda i: (i, 0)),
        out_shape=jax.ShapeDtypeStruct((nb * (BN // 2), 2 * DIM), jnp.float32),
        compiler_params=pltpu.CompilerParams(
            dimension_semantics=("parallel",),
        ),
    )(tabT)


def _packed_offsets(ids, B, L, nw, steps2):
    t = ids.T.astype(jnp.int32)
    a = jnp.transpose(t.reshape(L, 2, B // 2), (0, 2, 1))
    blk = a // BN
    h = (a % BN) // (BN // 2)
    p = a % (BN // 2)
    v0 = blk * (4 * (BN // 2)) + p * 4 + h * 2
    ids2 = jnp.stack([v0, v0 + 1], axis=-1)
    return ids2.reshape(nw, steps2, G)


def _sc_gather_one(idx, packed, n_idx, nw, steps2):
    view = packed.reshape(packed.shape[0] * 4, 32)
    mesh = plsc.VectorSubcoreMesh(core_axis_name="c", subcore_axis_name="s")
    nc = mesh.num_cores

    def body(idx_hbm, tab_hbm, out_hbm, sidx, buf, gsem):
        wid = lax.axis_index("s") * nc + lax.axis_index("c")
        pltpu.sync_copy(idx_hbm.at[wid], sidx)
        row0 = wid * (steps2 * G)

        pltpu.async_copy(tab_hbm.at[sidx.at[0]], buf.at[0], gsem)

        def step(j, carry):
            slot = lax.rem(j, 2)
            nxt = lax.rem(j + 1, 2)
            pltpu.make_async_copy(tab_hbm.at[sidx.at[j]], buf.at[slot],
                                  gsem).wait()

            @pl.when(j + 1 < steps2)
            def _():
                pltpu.async_copy(tab_hbm.at[sidx.at[j + 1]], buf.at[nxt],
                                 gsem)

            pltpu.sync_copy(buf.at[slot],
                            out_hbm.at[pl.ds(row0 + j * G, G)])
            return carry

        lax.fori_loop(0, steps2, step, 0)

    f = pl.kernel(
        body,
        out_type=jax.ShapeDtypeStruct((n_idx, 32), jnp.float32),
        mesh=mesh,
        scratch_types=[
            pltpu.VMEM((steps2, G), jnp.int32),
            pltpu.VMEM((2, G, 32), jnp.float32),
            pltpu.SemaphoreType.DMA,
        ],
        compiler_params=pltpu.CompilerParams(use_tc_tiling_on_sc=False),
    )
    return f(idx, view)


def _ln_mlp_half_t(x, w1g, b1bt, w2, b2t):
    mu = jnp.mean(x, axis=-1, keepdims=True)
    xc = x - mu
    var = jnp.mean(xc * xc, axis=-1, keepdims=True)
    y = xc * lax.rsqrt(var + 1e-5)
    ht = lax.dot_general(w1g, y, (((0,), (1,)), ((), ())),
                         preferred_element_type=jnp.float32) + b1bt
    ht = 0.5 * ht * (1.0 + lax.erf(ht * (1.0 / math.sqrt(2.0))))
    return lax.dot_general(w2, ht, (((0,), (0,)), ((), ())),
                           preferred_element_type=jnp.float32) + b2t


def _mlp_body(es_ref, eg_ref, w1g_ref, b1bt_ref, w2_ref, b2t_ref, o_ref):
    x = es_ref[...] + eg_ref[...]
    z0t = _ln_mlp_half_t(x[:, :DIM], w1g_ref[...], b1bt_ref[...], w2_ref[...],
                         b2t_ref[...])
    z1t = _ln_mlp_half_t(x[:, DIM:], w1g_ref[...], b1bt_ref[...], w2_ref[...],
                         b2t_ref[...])
    r = x.shape[0]
    o_ref[0, :, 0:r] = z0t
    o_ref[0, :, r:2 * r] = z1t


def _tc_mlp(es, eg, gamma, beta, W1, b1, W2, b2, B, L):
    n_wide = es.shape[0] // 2
    wide_per_l = B // 2
    esw = es.reshape(n_wide, 2 * DIM)
    egw = eg.reshape(n_wide, 2 * DIM)
    w1g = gamma[:, None] * W1
    b1bt = (beta @ W1 + b1).reshape(HID, 1)
    b2t = b2.reshape(DIM, 1)
    full = lambda shape: pl.BlockSpec(shape, lambda i: (0,) * len(shape))
    out = pl.pallas_call(
        _mlp_body,
        grid=(L,),
        in_specs=[
            pl.BlockSpec((wide_per_l, 2 * DIM), lambda i: (i, 0)),
            pl.BlockSpec((wide_per_l, 2 * DIM), lambda i: (i, 0)),
            full((DIM, HID)),
            full((HID, 1)),
            full((HID, DIM)),
            full((DIM, 1)),
        ],
        out_specs=pl.BlockSpec((1, DIM, B), lambda i: (i, 0, 0)),
        out_shape=jax.ShapeDtypeStruct((L, DIM, B), jnp.float32),
        compiler_params=pltpu.CompilerParams(
            dimension_semantics=("parallel",),
        ),
    )(esw, egw, w1g, b1bt, W2, b2t)
    return jnp.transpose(out, (2, 0, 1))


def kernel(station_ids, geometry_ids, station_table, geometry_table, gamma,
           beta, W1, b1, W2, b2):
    B, L = station_ids.shape
    n = B * L
    nw = 32
    steps2 = 2 * n // (nw * G)
    assert steps2 * nw * G == 2 * n

    ids2_g = _packed_offsets(geometry_ids, B, L, nw, steps2)
    ids2_s = _packed_offsets(station_ids, B, L, nw, steps2)
    gpack = _transpose_pack(geometry_table.T)
    spack = _transpose_pack(station_table.T)
    eg = _sc_gather_one(ids2_g, gpack, 2 * n, nw, steps2).reshape(n, DIM)
    es = _sc_gather_one(ids2_s, spack, 2 * n, nw, steps2).reshape(n, DIM)
    return _tc_mlp(es, eg, gamma, beta, W1, b1, W2, b2, B, L)
